# Initial kernel scaffold; baseline (speedup 1.0000x reference)
import jax
import jax.numpy as jnp
from jax.experimental import pallas as pl


def kernel(feat, label):
    raise NotImplementedError("write your pallas kernel here")


# speedup vs baseline: 82.1648x; 82.1648x over previous
import jax
import jax.numpy as jnp
from jax import lax
from jax.experimental import pallas as pl
from jax.experimental.pallas import tpu as pltpu
from jax.experimental.pallas import tpu_sc as plsc

B = 8
C = 4
N = 512 * 512
NC = 2
NS = 16
L = 16
BPC = B // NC
CH = N // NS
NV = CH // L
NACC = 24

DELTA_V = 0.5
DELTA_D = 3.0


def _rsqrt_pos(x, iters=3):
    i = plsc.bitcast(x, jnp.int32)
    y = plsc.bitcast(jnp.int32(0x5F3759DF) - (i >> 1), jnp.float32)
    xh = x * 0.5
    for _ in range(iters):
        y = y * (1.5 - xh * y * y)
    return y


def _sqrt_guard(x):
    return x * _rsqrt_pos(jnp.maximum(x, 1e-30))


def _body(feat_hbm, lab_hbm, out_hbm,
          lab_v, feat_v, stage_v, gath_v, tab_v, res_v, part_s, var_s, sem):
    cid = lax.axis_index("c")
    sid = lax.axis_index("s")
    off = sid * CH
    iota = lax.iota(jnp.int32, L)
    zeros = jnp.zeros((L,), jnp.float32)

    lv_run = zeros
    ld_run = zeros
    lr_run = zeros

    for u in range(BPC):
        bi = cid * BPC + u

        cps = [pltpu.async_copy(lab_hbm.at[bi, pl.ds(off, CH)], lab_v, sem)]
        for ci in range(C):
            cps.append(pltpu.async_copy(
                feat_hbm.at[bi, ci, pl.ds(off, CH)], feat_v.at[ci], sem))
        for cp in cps:
            cp.wait()

        def p1(j, carry):
            accs = list(carry)
            base = j * L
            lab = lab_v[pl.ds(base, L)]
            fs = [feat_v[ci, pl.ds(base, L)] for ci in range(C)]
            o = 0
            for k in range(1, 5):
                m = lab == k
                accs[o] = accs[o] + jnp.where(m, 1.0, 0.0)
                o += 1
                for ci in range(C):
                    accs[o] = accs[o] + jnp.where(m, fs[ci], 0.0)
                    o += 1
            for ci in range(C):
                accs[o] = accs[o] + fs[ci]
                o += 1
            return tuple(accs)

        accs = lax.fori_loop(0, NV, p1, (zeros,) * NACC)

        for a in range(NACC):
            stage_v[a] = accs[a]
        pltpu.sync_copy(stage_v, part_s.at[u, sid])
        plsc.subcore_barrier()
        pltpu.sync_copy(part_s.at[u], gath_v)

        def red(t, carry):
            return tuple(carry[a] + gath_v[t, a] for a in range(NACC))

        tot = lax.fori_loop(0, NS, red, (zeros,) * NACC)

        cnt = [None] * 5
        seg = [[None] * C for _ in range(5)]
        o = 0
        for k in range(1, 5):
            cnt[k] = jnp.sum(tot[o])
            o += 1
            for ci in range(C):
                seg[k][ci] = jnp.sum(tot[o])
                o += 1
        tot_c = [jnp.sum(tot[o + ci]) for ci in range(C)]
        cnt[0] = float(N) - (cnt[1] + cnt[2] + cnt[3] + cnt[4])
        for ci in range(C):
            seg[0][ci] = tot_c[ci] - (seg[1][ci] + seg[2][ci]
                                      + seg[3][ci] + seg[4][ci])

        cnt_vec = zeros
        for k in range(5):
            cnt_vec = jnp.where(iota == k, cnt[k], cnt_vec)
        present = cnt_vec > 0.0
        presf = jnp.where(present, 1.0, 0.0)
        safe = jnp.where(present, cnt_vec, 1.0)
        inv_vec = 1.0 / safe
        K = jnp.sum(presf)
        invK = 1.0 / jnp.broadcast_to(K, (L,))

        mu_vecs = []
        for ci in range(C):
            sv = zeros
            for k in range(5):
                sv = jnp.where(iota == k, seg[k][ci], sv)
            mu_vecs.append(sv * inv_vec)
            tab_v[ci] = mu_vecs[ci]
        tab_v[4] = inv_vec

        d2r = mu_vecs[0] * mu_vecs[0]
        for ci in range(1, C):
            d2r = d2r + mu_vecs[ci] * mu_vecs[ci]
        lr_run = lr_run + jnp.where(present, _sqrt_guard(d2r), 0.0) * invK

        acc_d = zeros
        acc_m = zeros
        for a in range(5):
            mu_a = [jnp.sum(jnp.where(iota == a, mu_vecs[ci], 0.0))
                    for ci in range(C)]
            pa = jnp.sum(jnp.where(iota == a, presf, 0.0))
            sabs = zeros
            d2 = zeros
            for ci in range(C):
                df = mu_vecs[ci] - mu_a[ci]
                sabs = sabs + jnp.abs(df)
                d2 = d2 + df * df
            mf = jnp.where((sabs != 0.0) & present, pa, 0.0)
            h = jnp.maximum(2.0 * DELTA_D - _sqrt_guard(d2), 0.0)
            acc_d = acc_d + h * h * mf
            acc_m = acc_m + mf
        Mtot = jnp.sum(acc_m)
        ld_run = ld_run + acc_d / jnp.broadcast_to(Mtot, (L,))

        c_idx = [jnp.full((L,), ci, jnp.int32) for ci in range(C)]
        w_idx = jnp.full((L,), 4, jnp.int32)

        def p2(j, acc):
            base = j * L
            lab = lab_v[pl.ds(base, L)]
            d2p = zeros
            for ci in range(C):
                g = plsc.load_gather(tab_v, [c_idx[ci], lab])
                t = feat_v[ci, pl.ds(base, L)] - g
                d2p = d2p + t * t
            w = plsc.load_gather(tab_v, [w_idx, lab])
            t = jnp.maximum(d2p, 0.0625)
            d = t * _rsqrt_pos(t)
            h = jnp.maximum(d - DELTA_V, 0.0)
            return acc + h * h * w

        accv = lax.fori_loop(0, NV, p2, zeros)
        lv_run = lv_run + accv * invK

    stage_v[0] = lv_run
    pltpu.sync_copy(stage_v.at[0], var_s.at[sid])
    plsc.subcore_barrier()

    @pl.when(sid == 0)
    def _():
        pltpu.sync_copy(var_s, gath_v.at[0, pl.ds(0, NS)])
        vsum = zeros
        for t in range(NS):
            vsum = vsum + gath_v[0, t]
        lv_tot = jnp.sum(vsum)
        ld_tot = jnp.sum(ld_run)
        lr_tot = jnp.sum(lr_run)
        res = jnp.where(iota == 0, lv_tot, zeros)
        res = jnp.where(iota == 1, ld_tot, res)
        res = jnp.where(iota == 2, lr_tot, res)
        res_v[...] = res
        pltpu.sync_copy(res_v, out_hbm.at[cid])


def _make_call():
    mesh = plsc.VectorSubcoreMesh(core_axis_name="c", subcore_axis_name="s",
                                  num_cores=NC, num_subcores=NS)
    return pl.kernel(
        _body,
        out_type=jax.ShapeDtypeStruct((NC, L), jnp.float32),
        mesh=mesh,
        compiler_params=pltpu.CompilerParams(needs_layout_passes=False,
                                             use_tc_tiling_on_sc=False),
        scratch_types=[
            pltpu.VMEM((CH,), jnp.int32),
            pltpu.VMEM((C, CH), jnp.float32),
            pltpu.VMEM((NACC, L), jnp.float32),
            pltpu.VMEM((NS, NACC, L), jnp.float32),
            pltpu.VMEM((5, L), jnp.float32),
            pltpu.VMEM((L,), jnp.float32),
            pltpu.VMEM_SHARED((BPC, NS, NACC, L), jnp.float32),
            pltpu.VMEM_SHARED((NS, L), jnp.float32),
            pltpu.SemaphoreType.DMA,
        ],
    )


def kernel(feat, label):
    feat_r = feat.reshape(B, C, N)
    lab_r = label.reshape(B, N).astype(jnp.int32)
    out = _make_call()(feat_r, lab_r)
    s = out[0] + out[1]
    lvr = 1.0 * s[0]
    ldr = 1.0 * s[1]
    lrr = 0.001 * s[2]
    loss = lvr + ldr + lrr
    return (loss / B, lvr / B, ldr / B, lrr / B)
